# baseline (speedup 1.0000x reference)
import functools

import jax
import jax.numpy as jnp
from jax import lax
from jax.experimental import pallas as pl
from jax.experimental.pallas import tpu as pltpu
from jax.experimental.pallas import tpu_sc as plsc

_TOPK = 8
_E = 64
_T = 16384
_NC = 2
_NS = 16
_L = 16
_NW = _NC * _NS
_TPW = _T // _NW


def _sigmoid(x):
    return 1.0 / (1.0 + jnp.exp(-x))


def _merge_top16(ka, va, kb, vb):
    gt = ka > kb
    eq = ka == kb
    km = jnp.maximum(ka, kb)
    vm = jnp.where(gt, va, vb)
    vm = jnp.where(eq, jnp.minimum(va, vb), vm)
    return km, vm


def _routing_body(logits_hbm, bias_hbm, idx_hbm, w_hbm,
                  slab, bias_v, idx_st, w_st):
    wid = lax.axis_index("s") * _NC + lax.axis_index("c")
    base = wid * _TPW

    pltpu.sync_copy(logits_hbm.at[pl.ds(base * _E, _TPW * _E)], slab)
    pltpu.sync_copy(bias_hbm, bias_v)

    lane = lax.iota(jnp.int32, _L)
    low8 = lane < _TOPK
    lane_mod8 = jnp.bitwise_and(lane, _TOPK - 1)
    bias_r = [bias_v[pl.ds(j * _L, _L)] for j in range(_E // _L)]
    vids = [lane + j * _L for j in range(_E // _L)]

    def body(t):
        sorted_kv = []
        for j in range(_E // _L):
            x = slab[pl.ds(t * _E + j * _L, _L)]
            k = _sigmoid(x) + bias_r[j]
            sorted_kv.append(
                plsc.sort_key_val(k, vids[j], descending=(j % 2 == 0)))
        k01, v01 = _merge_top16(sorted_kv[0][0], sorted_kv[0][1],
                                sorted_kv[1][0], sorted_kv[1][1])
        k23, v23 = _merge_top16(sorted_kv[2][0], sorted_kv[2][1],
                                sorted_kv[3][0], sorted_kv[3][1])
        k01, v01 = plsc.sort_key_val(k01, v01, descending=True)
        k23, v23 = plsc.sort_key_val(k23, v23, descending=False)
        kt, vt = _merge_top16(k01, v01, k23, v23)
        _, top_ids = plsc.sort_key_val(kt, vt, descending=True)

        xg = plsc.load_gather(slab, [t * _E + top_ids])
        sg = jnp.where(low8, _sigmoid(xg), 0.0)
        total = jnp.broadcast_to(jnp.sum(sg), (_L,)) + 1e-20
        w = sg / total

        out_pos = t * _TOPK + lane_mod8
        plsc.store_scatter(idx_st, [out_pos], top_ids, mask=low8)
        plsc.store_scatter(w_st, [out_pos], w, mask=low8)

    plsc.parallel_loop(0, _TPW, 1, unroll=8)(body)

    pltpu.sync_copy(idx_st, idx_hbm.at[pl.ds(base * _TOPK, _TPW * _TOPK)])
    pltpu.sync_copy(w_st, w_hbm.at[pl.ds(base * _TOPK, _TPW * _TOPK)])


@jax.jit
def kernel(router_logits, e_score_correction_bias):
    routing = functools.partial(
        pl.kernel,
        out_type=(
            jax.ShapeDtypeStruct((_T * _TOPK,), jnp.int32),
            jax.ShapeDtypeStruct((_T * _TOPK,), jnp.float32),
        ),
        mesh=plsc.VectorSubcoreMesh(
            core_axis_name="c", subcore_axis_name="s",
            num_cores=_NC, num_subcores=_NS),
        scratch_types=[
            pltpu.VMEM((_TPW * _E,), jnp.float32),
            pltpu.VMEM((_E,), jnp.float32),
            pltpu.VMEM((_TPW * _TOPK,), jnp.int32),
            pltpu.VMEM((_TPW * _TOPK,), jnp.float32),
        ],
        compiler_params=pltpu.CompilerParams(needs_layout_passes=False),
    )(_routing_body)
    idx_flat, w_flat = routing(
        router_logits.reshape(_T * _E), e_score_correction_bias)
    return (idx_flat.reshape(_T, _TOPK), w_flat.reshape(_T, _TOPK))


# speedup vs baseline: 1.0569x; 1.0569x over previous
import functools

import jax
import jax.numpy as jnp
from jax import lax
from jax.experimental import pallas as pl
from jax.experimental.pallas import tpu as pltpu
from jax.experimental.pallas import tpu_sc as plsc

_TOPK = 8
_E = 64
_T = 16384
_NC = 2
_NS = 16
_L = 16
_NW = _NC * _NS
_TPW = _T // _NW


def _sigmoid(x):
    return 1.0 / (1.0 + jnp.exp(-x))


def _merge_top16(ka, va, kb, vb):
    gt = ka > kb
    eq = ka == kb
    km = jnp.maximum(ka, kb)
    vm = jnp.where(gt, va, vb)
    vm = jnp.where(eq, jnp.minimum(va, vb), vm)
    return km, vm


def _routing_body(logits_hbm, bias_hbm, idx_hbm, w_hbm,
                  slab, bias_v, idx_st, w_st):
    wid = lax.axis_index("s") * _NC + lax.axis_index("c")
    base = wid * _TPW

    pltpu.sync_copy(logits_hbm.at[pl.ds(base * _E, _TPW * _E)], slab)
    pltpu.sync_copy(bias_hbm, bias_v)

    lane = lax.iota(jnp.int32, _L)
    low8 = lane < _TOPK
    lane_mod8 = jnp.bitwise_and(lane, _TOPK - 1)
    bias_r = [bias_v[pl.ds(j * _L, _L)] for j in range(_E // _L)]
    vids = [lane + j * _L for j in range(_E // _L)]

    def body(t):
        sorted_kv = []
        for j in range(_E // _L):
            x = slab[pl.ds(t * _E + j * _L, _L)]
            k = _sigmoid(x) + bias_r[j]
            sorted_kv.append(
                plsc.sort_key_val(k, vids[j], descending=(j % 2 == 0)))
        k01, v01 = _merge_top16(sorted_kv[0][0], sorted_kv[0][1],
                                sorted_kv[1][0], sorted_kv[1][1])
        k23, v23 = _merge_top16(sorted_kv[2][0], sorted_kv[2][1],
                                sorted_kv[3][0], sorted_kv[3][1])
        k01, v01 = plsc.sort_key_val(k01, v01, descending=True)
        k23, v23 = plsc.sort_key_val(k23, v23, descending=False)
        kt, vt = _merge_top16(k01, v01, k23, v23)
        _, top_ids = plsc.sort_key_val(kt, vt, descending=True)

        xg = plsc.load_gather(slab, [t * _E + top_ids])
        sg = jnp.where(low8, _sigmoid(xg), 0.0)
        total = jnp.broadcast_to(jnp.sum(sg), (_L,)) + 1e-20
        w = sg / total

        out_pos = t * _TOPK + lane_mod8
        plsc.store_scatter(idx_st, [out_pos], top_ids, mask=low8)
        plsc.store_scatter(w_st, [out_pos], w, mask=low8)

    plsc.parallel_loop(0, _TPW, 1, unroll=2)(body)

    pltpu.sync_copy(idx_st, idx_hbm.at[pl.ds(base * _TOPK, _TPW * _TOPK)])
    pltpu.sync_copy(w_st, w_hbm.at[pl.ds(base * _TOPK, _TPW * _TOPK)])


@jax.jit
def kernel(router_logits, e_score_correction_bias):
    routing = functools.partial(
        pl.kernel,
        out_type=(
            jax.ShapeDtypeStruct((_T * _TOPK,), jnp.int32),
            jax.ShapeDtypeStruct((_T * _TOPK,), jnp.float32),
        ),
        mesh=plsc.VectorSubcoreMesh(
            core_axis_name="c", subcore_axis_name="s",
            num_cores=_NC, num_subcores=_NS),
        scratch_types=[
            pltpu.VMEM((_TPW * _E,), jnp.float32),
            pltpu.VMEM((_E,), jnp.float32),
            pltpu.VMEM((_TPW * _TOPK,), jnp.int32),
            pltpu.VMEM((_TPW * _TOPK,), jnp.float32),
        ],
        compiler_params=pltpu.CompilerParams(needs_layout_passes=False),
    )(_routing_body)
    idx_flat, w_flat = routing(
        router_logits.reshape(_T * _E), e_score_correction_bias)
    return (idx_flat.reshape(_T, _TOPK), w_flat.reshape(_T, _TOPK))
